# baseline (speedup 1.0000x reference)
import functools

import jax
import jax.numpy as jnp
from jax import lax
from jax.experimental import pallas as pl
from jax.experimental.pallas import tpu as pltpu
from jax.experimental.pallas import tpu_sc as plsc

BATCH = 16384
EMBED_DIM = 64

_info = plsc.get_sparse_core_info()
_NC, _NS = _info.num_cores, _info.num_subcores
_NW = _NC * _NS
_B_PER_W = BATCH // _NW

_mesh = plsc.VectorSubcoreMesh(core_axis_name="c", subcore_axis_name="s")


@functools.partial(
    pl.kernel,
    mesh=_mesh,
    compiler_params=pltpu.CompilerParams(use_tc_tiling_on_sc=False),
    out_type=(
        jax.ShapeDtypeStruct((BATCH, EMBED_DIM), jnp.float32),
        jax.ShapeDtypeStruct((BATCH, EMBED_DIM), jnp.float32),
    ),
    scratch_types=[
        pltpu.VMEM((_B_PER_W,), jnp.int32),
        pltpu.VMEM((_B_PER_W,), jnp.int32),
        pltpu.VMEM((_B_PER_W, EMBED_DIM), jnp.float32),
        pltpu.VMEM((_B_PER_W, EMBED_DIM), jnp.float32),
        pltpu.SemaphoreType.DMA,
        pltpu.SemaphoreType.DMA,
        pltpu.SemaphoreType.DMA,
        pltpu.SemaphoreType.DMA,
    ],
)
def _bpr_lookup(user_hbm, item_hbm, utab_hbm, itab_hbm, uout_hbm, iout_hbm,
                uidx_v, iidx_v, urows_v, irows_v, sem_u, sem_i, sem_uo, sem_io):
    wid = lax.axis_index("s") * _NC + lax.axis_index("c")
    base = wid * _B_PER_W
    pltpu.sync_copy(user_hbm.at[pl.ds(base, _B_PER_W)], uidx_v)
    pltpu.sync_copy(item_hbm.at[pl.ds(base, _B_PER_W)], iidx_v)
    u_gather = pltpu.async_copy(utab_hbm.at[uidx_v], urows_v, sem_u)
    i_gather = pltpu.async_copy(itab_hbm.at[iidx_v], irows_v, sem_i)
    u_gather.wait()
    u_out = pltpu.async_copy(urows_v, uout_hbm.at[pl.ds(base, _B_PER_W)], sem_uo)
    i_gather.wait()
    i_out = pltpu.async_copy(irows_v, iout_hbm.at[pl.ds(base, _B_PER_W)], sem_io)
    u_out.wait()
    i_out.wait()


def kernel(user, item, user_table, item_table):
    return _bpr_lookup(user, item, user_table, item_table)
